# baseline (speedup 1.0000x reference)
import functools

import jax
import jax.numpy as jnp
from jax import lax
from jax.experimental import pallas as pl
from jax.experimental.pallas import tpu as pltpu
from jax.experimental.pallas import tpu_sc as plsc

_B = 4096
_L = 200
_EMB = 64
_V = 1000000
_NW = 32
_VP = _V // 2
_NBLK = (_V // 128)
_BPW = _NBLK // _NW
_REM = _NBLK - _BPW * _NW
_PAIRS_A = _BPW // 2
_BB = _B // _NW
_PAIRS_B = _L // 2

_mesh = plsc.VectorSubcoreMesh(core_axis_name="c", subcore_axis_name="s")
_params = pltpu.CompilerParams(use_tc_tiling_on_sc=True, needs_layout_passes=False)


def _iota16():
    return lax.iota(jnp.int32, 16)


@functools.partial(
    pl.kernel,
    mesh=_mesh,
    out_type=jax.ShapeDtypeStruct((_VP, 128), jnp.float32),
    scratch_types=[
        pltpu.VMEM((2, _EMB, 128), jnp.float32),
        pltpu.VMEM((2, _EMB, 128), jnp.float32),
        pltpu.VMEM((32, 128), jnp.float32),
        pltpu.SemaphoreType.DMA,
        pltpu.SemaphoreType.DMA,
    ],
    compiler_params=_params,
)
def _transpose_table(tT_hbm, tail_hbm, tlin_hbm, inv, obuf, tailv, isem, osem):
    wid = lax.axis_index("s") * 2 + lax.axis_index("c")
    base = wid * _BPW
    it16 = _iota16()

    def _start_in(j, b):
        pltpu.async_copy(tT_hbm.at[:, pl.ds(j * 128, 128)], inv.at[b], isem)

    def _wait_in(b):
        pltpu.make_async_copy(tT_hbm.at[:, pl.ds(0, 128)], inv.at[b], isem).wait()

    def _start_out(j, b):
        pltpu.async_copy(obuf.at[b], tlin_hbm.at[pl.ds(j * 64, 64)], osem)

    def _wait_out(b):
        pltpu.make_async_copy(obuf.at[b], tlin_hbm.at[pl.ds(0, 64)], osem).wait()

    def _vpu_transpose(b):
        def pbody(p, carry):
            for k in range(8):
                rows = it16 + ((16 * k) % 64)
                cols = jnp.full((16,), 2 * p + (1 if k >= 4 else 0), jnp.int32)
                val = plsc.load_gather(inv.at[b], [rows, cols])
                obuf[b, p, pl.ds(16 * k, 16)] = val
            return carry

        lax.fori_loop(0, 64, pbody, 0)

    _start_in(base, 0)

    def body(u, carry):
        t0 = base + 2 * u
        _wait_in(0)
        _start_in(t0 + 1, 1)

        @pl.when(u > 0)
        def _():
            _wait_out(0)

        _vpu_transpose(0)
        _start_out(t0, 0)

        _wait_in(1)

        @pl.when(u + 1 < _PAIRS_A)
        def _():
            _start_in(t0 + 2, 0)

        @pl.when(u > 0)
        def _():
            _wait_out(1)

        _vpu_transpose(1)
        _start_out(t0 + 1, 1)
        return carry

    lax.fori_loop(0, _PAIRS_A, body, 0)
    _wait_out(0)
    _wait_out(1)

    @pl.when(wid < _REM)
    def _():
        jx = _NBLK - _REM + wid
        _start_in(jx, 0)
        _wait_in(0)
        _vpu_transpose(0)
        _start_out(jx, 0)
        _wait_out(0)

    @pl.when(wid == _NW - 1)
    def _():
        pltpu.sync_copy(tail_hbm, tailv)
        pltpu.sync_copy(tailv, tlin_hbm.at[pl.ds(_NBLK * 64, 32)])


@functools.partial(
    pl.kernel,
    mesh=_mesh,
    out_type=jax.ShapeDtypeStruct((_L, _EMB, _B), jnp.float32),
    scratch_types=[
        pltpu.VMEM((_L, 128), jnp.int32),
        pltpu.VMEM((2, 128), jnp.int32),
        pltpu.VMEM((2, 128, 128), jnp.float32),
        pltpu.VMEM((2, _EMB, 128), jnp.float32),
        pltpu.SemaphoreType.DMA,
        pltpu.SemaphoreType.DMA,
    ],
    compiler_params=_params,
)
def _emb_gather(seqT_hbm, tlin_hbm, out_hbm, idxv, pidx, gbuf, tbuf, gsem, osem):
    wid = lax.axis_index("s") * 2 + lax.axis_index("c")
    bcol = wid * _BB
    it16 = _iota16()

    pltpu.sync_copy(seqT_hbm.at[:, pl.ds(bcol, _BB)], idxv)

    def _make_pidx(l, b):
        for k in range(8):
            v = idxv[l, pl.ds(16 * k, 16)]
            pidx[b, pl.ds(16 * k, 16)] = lax.shift_right_logical(v, 1)

    def _start_gather(b):
        pltpu.async_copy(tlin_hbm.at[pidx.at[b]], gbuf.at[b], gsem)

    def _wait_gather(b):
        pltpu.make_async_copy(tlin_hbm.at[pidx.at[b]], gbuf.at[b], gsem).wait()

    def _start_out(l, b):
        pltpu.async_copy(tbuf.at[b], out_hbm.at[l, :, pl.ds(bcol, _BB)], osem)

    def _wait_out(b):
        pltpu.make_async_copy(tbuf.at[b], out_hbm.at[0, :, pl.ds(bcol, _BB)], osem).wait()

    def _vpu_transpose(l, b):
        colbases = []
        for k in range(8):
            v = idxv[l, pl.ds(16 * k, 16)]
            colbases.append((v & 1) * 64)
        rowvecs = [it16 + 16 * k for k in range(8)]

        def cbody(c, carry):
            cb = carry
            for k in range(8):
                val = plsc.load_gather(gbuf.at[b], [rowvecs[k], cb[k] + c])
                tbuf[b, c, pl.ds(16 * k, 16)] = val
            return cb

        lax.fori_loop(0, _EMB, cbody, tuple(colbases))

    _make_pidx(0, 0)
    _start_gather(0)

    def body(u, carry):
        l0 = 2 * u
        _wait_gather(0)
        _make_pidx(l0 + 1, 1)
        _start_gather(1)

        @pl.when(u > 0)
        def _():
            _wait_out(0)

        _vpu_transpose(l0, 0)
        _start_out(l0, 0)

        _wait_gather(1)

        @pl.when(u + 1 < _PAIRS_B)
        def _():
            _make_pidx(l0 + 2, 0)
            _start_gather(0)

        @pl.when(u > 0)
        def _():
            _wait_out(1)

        _vpu_transpose(l0 + 1, 1)
        _start_out(l0 + 1, 1)
        return carry

    lax.fori_loop(0, _PAIRS_B, body, 0)
    _wait_out(0)
    _wait_out(1)


def kernel(sequences, token_embedding_matrix):
    tT = jnp.transpose(token_embedding_matrix)
    tail = jnp.reshape(token_embedding_matrix[_NBLK * 128:, :], (32, 128))
    tlin = _transpose_table(tT, tail)
    seqT = jnp.transpose(sequences.astype(jnp.int32))
    outT = _emb_gather(seqT, tlin)
    emb = jnp.transpose(outT, (2, 0, 1))
    return emb, _L


# speedup vs baseline: 1.8788x; 1.8788x over previous
import functools

import jax
import jax.numpy as jnp
from jax import lax
from jax.experimental import pallas as pl
from jax.experimental.pallas import tpu as pltpu
from jax.experimental.pallas import tpu_sc as plsc

_B = 4096
_L = 200
_EMB = 64
_V = 1000000
_NW = 32
_VP = _V // 2
_NBLK = (_V // 128)
_BPW = _NBLK // _NW
_REM = _NBLK - _BPW * _NW
_PAIRS_A = _BPW // 2
_BB = _B // _NW
_PAIRS_B = _L // 2

_mesh = plsc.VectorSubcoreMesh(core_axis_name="c", subcore_axis_name="s")
_params = pltpu.CompilerParams(use_tc_tiling_on_sc=True, needs_layout_passes=False)


def _iota16():
    return lax.iota(jnp.int32, 16)


@functools.partial(
    pl.kernel,
    mesh=_mesh,
    out_type=jax.ShapeDtypeStruct((_VP, 128), jnp.float32),
    scratch_types=[
        pltpu.VMEM((2, _EMB, 128), jnp.float32),
        pltpu.VMEM((2, _EMB, 128), jnp.float32),
        pltpu.VMEM((32, 128), jnp.float32),
        pltpu.SemaphoreType.DMA,
        pltpu.SemaphoreType.DMA,
    ],
    compiler_params=_params,
)
def _transpose_table(tT_hbm, tail_hbm, tlin_hbm, inv, obuf, tailv, isem, osem):
    wid = lax.axis_index("s") * 2 + lax.axis_index("c")
    base = wid * _BPW
    it16 = _iota16()

    def _start_in(j, b):
        pltpu.async_copy(tT_hbm.at[:, pl.ds(j * 128, 128)], inv.at[b], isem)

    def _wait_in(b):
        pltpu.make_async_copy(tT_hbm.at[:, pl.ds(0, 128)], inv.at[b], isem).wait()

    def _start_out(j, b):
        pltpu.async_copy(obuf.at[b], tlin_hbm.at[pl.ds(j * 64, 64)], osem)

    def _wait_out(b):
        pltpu.make_async_copy(obuf.at[b], tlin_hbm.at[pl.ds(0, 64)], osem).wait()

    def _vpu_transpose(b):
        @plsc.parallel_loop(0, 64, unroll=4)
        def pbody(p):
            for k in range(8):
                rows = it16 + ((16 * k) % 64)
                cols = jnp.full((16,), 2 * p + (1 if k >= 4 else 0), jnp.int32)
                val = plsc.load_gather(inv.at[b], [rows, cols])
                obuf[b, p, pl.ds(16 * k, 16)] = val

    _start_in(base, 0)

    def body(u, carry):
        t0 = base + 2 * u
        _wait_in(0)
        _start_in(t0 + 1, 1)

        @pl.when(u > 0)
        def _():
            _wait_out(0)

        _vpu_transpose(0)
        _start_out(t0, 0)

        _wait_in(1)

        @pl.when(u + 1 < _PAIRS_A)
        def _():
            _start_in(t0 + 2, 0)

        @pl.when(u > 0)
        def _():
            _wait_out(1)

        _vpu_transpose(1)
        _start_out(t0 + 1, 1)
        return carry

    lax.fori_loop(0, _PAIRS_A, body, 0)
    _wait_out(0)
    _wait_out(1)

    @pl.when(wid < _REM)
    def _():
        jx = _NBLK - _REM + wid
        _start_in(jx, 0)
        _wait_in(0)
        _vpu_transpose(0)
        _start_out(jx, 0)
        _wait_out(0)

    @pl.when(wid == _NW - 1)
    def _():
        pltpu.sync_copy(tail_hbm, tailv)
        pltpu.sync_copy(tailv, tlin_hbm.at[pl.ds(_NBLK * 64, 32)])


@functools.partial(
    pl.kernel,
    mesh=_mesh,
    out_type=jax.ShapeDtypeStruct((_L, _EMB, _B), jnp.float32),
    scratch_types=[
        pltpu.VMEM((_L, 128), jnp.int32),
        pltpu.VMEM((2, 128), jnp.int32),
        pltpu.VMEM((2, 128, 128), jnp.float32),
        pltpu.VMEM((2, _EMB, 128), jnp.float32),
        pltpu.SemaphoreType.DMA,
        pltpu.SemaphoreType.DMA,
    ],
    compiler_params=_params,
)
def _emb_gather(seqT_hbm, tlin_hbm, out_hbm, idxv, pidx, gbuf, tbuf, gsem, osem):
    wid = lax.axis_index("s") * 2 + lax.axis_index("c")
    bcol = wid * _BB
    it16 = _iota16()

    pltpu.sync_copy(seqT_hbm.at[:, pl.ds(bcol, _BB)], idxv)

    def _make_pidx(l, b):
        for k in range(8):
            v = idxv[l, pl.ds(16 * k, 16)]
            pidx[b, pl.ds(16 * k, 16)] = lax.shift_right_logical(v, 1)

    def _start_gather(b):
        pltpu.async_copy(tlin_hbm.at[pidx.at[b]], gbuf.at[b], gsem)

    def _wait_gather(b):
        pltpu.make_async_copy(tlin_hbm.at[pidx.at[b]], gbuf.at[b], gsem).wait()

    def _start_out(l, b):
        pltpu.async_copy(tbuf.at[b], out_hbm.at[l, :, pl.ds(bcol, _BB)], osem)

    def _wait_out(b):
        pltpu.make_async_copy(tbuf.at[b], out_hbm.at[0, :, pl.ds(bcol, _BB)], osem).wait()

    def _vpu_transpose(l, b):
        colbases = []
        for k in range(8):
            v = idxv[l, pl.ds(16 * k, 16)]
            colbases.append((v & 1) * 64)
        rowvecs = [it16 + 16 * k for k in range(8)]

        @plsc.parallel_loop(0, _EMB, unroll=4)
        def cbody(c):
            for k in range(8):
                val = plsc.load_gather(gbuf.at[b], [rowvecs[k], colbases[k] + c])
                tbuf[b, c, pl.ds(16 * k, 16)] = val

    _make_pidx(0, 0)
    _start_gather(0)

    def body(u, carry):
        l0 = 2 * u
        _wait_gather(0)
        _make_pidx(l0 + 1, 1)
        _start_gather(1)

        @pl.when(u > 0)
        def _():
            _wait_out(0)

        _vpu_transpose(l0, 0)
        _start_out(l0, 0)

        _wait_gather(1)

        @pl.when(u + 1 < _PAIRS_B)
        def _():
            _make_pidx(l0 + 2, 0)
            _start_gather(0)

        @pl.when(u > 0)
        def _():
            _wait_out(1)

        _vpu_transpose(l0 + 1, 1)
        _start_out(l0 + 1, 1)
        return carry

    lax.fori_loop(0, _PAIRS_B, body, 0)
    _wait_out(0)
    _wait_out(1)


def kernel(sequences, token_embedding_matrix):
    tT = jnp.transpose(token_embedding_matrix)
    tail = jnp.reshape(token_embedding_matrix[_NBLK * 128:, :], (32, 128))
    tlin = _transpose_table(tT, tail)
    seqT = jnp.transpose(sequences.astype(jnp.int32))
    outT = _emb_gather(seqT, tlin)
    emb = jnp.transpose(outT, (2, 0, 1))
    return emb, _L
